# baseline (speedup 1.0000x reference)
import functools

import jax
import jax.numpy as jnp
from jax import lax
from jax.experimental import pallas as pl
from jax.experimental.pallas import tpu as pltpu
from jax.experimental.pallas import tpu_sc as plsc

LANES = 16
CHUNK = 256
NSUB = 16
BLK = 2000
SC_FRAC = 0.5


def _make_seg_sum_sc(n_rows: int, d: int, nseg: int, split: int):
  nlane_blocks = d // LANES
  pt = n_rows // NSUB
  bits = []
  b = 1
  while b <= pt:
    b *= 2
  while b >= 1:
    bits.append(b)
    b //= 2
  mesh = plsc.VectorSubcoreMesh(core_axis_name="c", subcore_axis_name="s")

  @functools.partial(
      pl.kernel,
      out_type=(
          jax.ShapeDtypeStruct((nseg, d), jnp.float32),
          jax.ShapeDtypeStruct((48,), jnp.int32),
          jax.ShapeDtypeStruct((2 * NSUB, 2 * LANES), jnp.int32),
      ),
      mesh=mesh,
      compiler_params=pltpu.CompilerParams(needs_layout_passes=False),
      scratch_types=[
          pltpu.VMEM((pt,), jnp.int32),
          pltpu.VMEM((CHUNK, d), jnp.float32),
          pltpu.VMEM((CHUNK, d), jnp.float32),
          pltpu.VMEM((48,), jnp.int32),
          pltpu.VMEM((2 * LANES,), jnp.int32),
          pltpu.VMEM((NSUB, 2 * LANES), jnp.int32),
          pltpu.VMEM((d,), jnp.float32),
          pltpu.SemaphoreType.DMA,
          pltpu.SemaphoreType.DMA,
      ],
  )
  def seg_sum(x_hbm, labels_hbm, out_hbm, bounds_hbm, stage_hbm,
              lab, buf0, buf1, bnd, cbuf, call, row, sem0, sem1):
    num_cores = jax.lax.axis_size("c")
    cid = lax.axis_index("c")
    tid = lax.axis_index("s")
    sid = tid * num_cores + cid

    pltpu.sync_copy(labels_hbm.at[pl.ds(tid * pt, pt)], lab)
    iota = lax.iota(jnp.int32, LANES)
    q0 = iota + 1
    q1 = iota + 17
    pos0 = jnp.zeros((LANES,), jnp.int32)
    pos1 = jnp.zeros((LANES,), jnp.int32)
    for bit in bits:
      cand0 = pos0 + bit
      cand1 = pos1 + bit
      v0 = plsc.load_gather(lab, [jnp.minimum(cand0, pt) - 1])
      v1 = plsc.load_gather(lab, [jnp.minimum(cand1, pt) - 1])
      pos0 = jnp.where((cand0 <= pt) & (v0 < q0), cand0, pos0)
      pos1 = jnp.where((cand1 <= pt) & (v1 < q1), cand1, pos1)
    cbuf[pl.ds(0, LANES)] = pos0
    cbuf[pl.ds(LANES, LANES)] = pos1
    pltpu.sync_copy(cbuf, stage_hbm.at[cid * NSUB + tid])
    plsc.subcore_barrier()
    pltpu.sync_copy(stage_hbm.at[pl.ds(cid * NSUB, NSUB)], call)
    g0 = jnp.zeros((LANES,), jnp.int32)
    g1 = jnp.zeros((LANES,), jnp.int32)
    for i in range(NSUB):
      g0 = g0 + call[i, pl.ds(0, LANES)]
      g1 = g1 + call[i, pl.ds(LANES, LANES)]
    bnd[pl.ds(0, LANES)] = g0
    bnd[pl.ds(LANES, LANES)] = g1

    @pl.when(sid == 0)
    def _():
      pltpu.sync_copy(bnd, bounds_hbm)

    off = jnp.maximum(sid - 1, 0)
    bv = bnd[pl.ds(off, LANES)]
    is0 = sid == 0
    r0 = jnp.minimum(jnp.where(is0, 0, bv[0]), split)
    r1 = jnp.minimum(jnp.where(is0, bv[0], bv[1]), split)
    base0 = (r0 // 8) * 8
    nch = lax.div(r1 - base0 + (CHUNK - 1), CHUNK)

    def win_start(j):
      return jnp.minimum(base0 + j * CHUNK, n_rows - CHUNK)

    def dma_start(j, buf, sem):
      pltpu.make_async_copy(x_hbm.at[pl.ds(win_start(j), CHUNK)],
                            buf, sem).start()

    def process(j, buf, sem, pfbuf, pfsem, accs):
      @pl.when(j + 1 < nch)
      def _():
        dma_start(j + 1, pfbuf, pfsem)

      @pl.when(j < nch)
      def _():
        pltpu.make_async_copy(x_hbm.at[pl.ds(0, CHUNK)], buf, sem).wait()

      w0 = win_start(j)
      a = jnp.maximum(r0, w0) - w0
      b = jnp.where(j < nch, jnp.minimum(r1, w0 + CHUNK) - w0, a)

      def row_body(r, accs):
        return tuple(accs[k] + buf[r, pl.ds(LANES * k, LANES)]
                     for k in range(nlane_blocks))

      return lax.fori_loop(a, b, row_body, accs)

    @pl.when(nch > 0)
    def _():
      dma_start(0, buf0, sem0)

    def pair_body(p, accs):
      j0 = 2 * p
      accs = process(j0, buf0, sem0, buf1, sem1, accs)
      accs = process(j0 + 1, buf1, sem1, buf0, sem0, accs)
      return accs

    zero = jnp.zeros((LANES,), jnp.float32)
    accs = lax.fori_loop(0, (nch + 1) // 2, pair_body,
                         tuple(zero for _ in range(nlane_blocks)))

    for k in range(nlane_blocks):
      row[pl.ds(LANES * k, LANES)] = accs[k]
    pltpu.sync_copy(row, out_hbm.at[sid])

  return seg_sum


def _make_tc_partial(nseg: int):
  def body(x_ref, lab_ref, out_ref):
    @pl.when(pl.program_id(0) == 0)
    def _():
      out_ref[...] = jnp.zeros_like(out_ref)

    lab = lab_ref[...].reshape(1, BLK)
    rows = lax.broadcasted_iota(jnp.int32, (nseg, BLK), 0)
    oh = (rows == lab).astype(jnp.float32)
    out_ref[...] += jnp.dot(oh, x_ref[...],
                            preferred_element_type=jnp.float32)

  return body


def _mlp_body(feats_ref, w1_ref, b1_ref, w2_ref, b2_ref, out_ref):
  h = jnp.dot(feats_ref[...], w1_ref[...],
              preferred_element_type=jnp.float32) + b1_ref[...]
  h = jnp.maximum(h, 0.0)
  logits = jnp.dot(h, w2_ref[...],
                   preferred_element_type=jnp.float32) + b2_ref[...]
  m = jnp.max(logits, axis=-1, keepdims=True)
  e = jnp.exp(logits - m)
  out_ref[...] = e / jnp.sum(e, axis=-1, keepdims=True)


def kernel(x, subject_labels, W1, b1, W2, b2):
  n, d = x.shape
  nseg = b2.shape[0]
  labels = subject_labels.astype(jnp.int32)

  nblk = n // BLK
  split = int(round(SC_FRAC * nblk)) * BLK
  offb = split // BLK

  feats_sc, bnd_out, _stage = _make_seg_sum_sc(n, d, nseg, split)(x, labels)

  feats_tc = pl.pallas_call(
      _make_tc_partial(nseg),
      grid=(nblk - offb,),
      in_specs=[
          pl.BlockSpec((BLK, d), lambda i: (offb + i, 0)),
          pl.BlockSpec((1, 1, BLK), lambda i: (offb + i, 0, 0)),
      ],
      out_specs=pl.BlockSpec((nseg, d), lambda i: (0, 0)),
      out_shape=jax.ShapeDtypeStruct((nseg, d), jnp.float32),
  )(x, labels.reshape(nblk, 1, BLK))

  bounds = jnp.concatenate([jnp.zeros((1,), jnp.int32), bnd_out[:nseg]])
  counts = bounds[1:] - bounds[:-1]
  feats = (feats_sc + feats_tc) / counts[:, None].astype(jnp.float32)

  probs = pl.pallas_call(
      _mlp_body,
      out_shape=jax.ShapeDtypeStruct((nseg, nseg), jnp.float32),
  )(feats, W1, b1.reshape(1, -1), W2, b2.reshape(1, -1))

  vals = jnp.arange(nseg, dtype=jnp.int32)
  present = counts > 0
  k = jnp.sum(present.astype(jnp.int32))
  ranks = jnp.cumsum(present.astype(jnp.int32)) - 1
  tgt = jnp.where(present, ranks, nseg - 1)
  compact = jnp.zeros((nseg,), jnp.int32).at[tgt].set(vals)
  uids = jnp.where(vals < k, compact, compact[0]).astype(subject_labels.dtype)

  return (probs, uids)


# speedup vs baseline: 1.1068x; 1.1068x over previous
import functools

import jax
import jax.numpy as jnp
from jax import lax
from jax.experimental import pallas as pl
from jax.experimental.pallas import tpu as pltpu
from jax.experimental.pallas import tpu_sc as plsc

LANES = 16
CHUNK = 256
NSUB = 16
NW = 32
BLK = 4000
SPLIT_UNIT = 32000
SC_FRAC = 0.7


def _make_seg_sum_sc(n_rows: int, d: int, nseg: int, split: int):
  nlane_blocks = d // LANES
  pt = n_rows // NSUB
  bits = []
  b = 1
  while b <= pt:
    b *= 2
  while b >= 1:
    bits.append(b)
    b //= 2
  mesh = plsc.VectorSubcoreMesh(core_axis_name="c", subcore_axis_name="s")

  @functools.partial(
      pl.kernel,
      out_type=(
          jax.ShapeDtypeStruct((NW, nseg, d), jnp.float32),
          jax.ShapeDtypeStruct((48,), jnp.int32),
          jax.ShapeDtypeStruct((2 * NSUB, 2 * LANES), jnp.int32),
      ),
      mesh=mesh,
      compiler_params=pltpu.CompilerParams(needs_layout_passes=False),
      scratch_types=[
          pltpu.VMEM((pt,), jnp.int32),
          pltpu.VMEM((CHUNK, d), jnp.float32),
          pltpu.VMEM((CHUNK, d), jnp.float32),
          pltpu.VMEM((48,), jnp.int32),
          pltpu.VMEM((2 * LANES,), jnp.int32),
          pltpu.VMEM((NSUB, 2 * LANES), jnp.int32),
          pltpu.VMEM((nseg, d), jnp.float32),
          pltpu.SemaphoreType.DMA,
          pltpu.SemaphoreType.DMA,
      ],
  )
  def seg_sum(x_hbm, labels_hbm, out_hbm, bounds_hbm, stage_hbm,
              lab, buf0, buf1, bnd, cbuf, call, acc2d, sem0, sem1):
    num_cores = jax.lax.axis_size("c")
    cid = lax.axis_index("c")
    tid = lax.axis_index("s")
    sid = tid * num_cores + cid

    pltpu.sync_copy(labels_hbm.at[pl.ds(tid * pt, pt)], lab)
    iota = lax.iota(jnp.int32, LANES)
    q0 = iota + 1
    q1 = iota + 17
    pos0 = jnp.zeros((LANES,), jnp.int32)
    pos1 = jnp.zeros((LANES,), jnp.int32)
    for bit in bits:
      cand0 = pos0 + bit
      cand1 = pos1 + bit
      v0 = plsc.load_gather(lab, [jnp.minimum(cand0, pt) - 1])
      v1 = plsc.load_gather(lab, [jnp.minimum(cand1, pt) - 1])
      pos0 = jnp.where((cand0 <= pt) & (v0 < q0), cand0, pos0)
      pos1 = jnp.where((cand1 <= pt) & (v1 < q1), cand1, pos1)
    cbuf[pl.ds(0, LANES)] = pos0
    cbuf[pl.ds(LANES, LANES)] = pos1
    pltpu.sync_copy(cbuf, stage_hbm.at[cid * NSUB + tid])
    plsc.subcore_barrier()
    pltpu.sync_copy(stage_hbm.at[pl.ds(cid * NSUB, NSUB)], call)
    g0 = jnp.zeros((LANES,), jnp.int32)
    g1 = jnp.zeros((LANES,), jnp.int32)
    for i in range(NSUB):
      g0 = g0 + call[i, pl.ds(0, LANES)]
      g1 = g1 + call[i, pl.ds(LANES, LANES)]
    bnd[pl.ds(0, LANES)] = g0
    bnd[pl.ds(LANES, LANES)] = g1

    @pl.when(sid == 0)
    def _():
      pltpu.sync_copy(bnd, bounds_hbm)

    share = split // NW
    lo = sid * share
    hi = lo + share

    def win_start(base0, j):
      return jnp.minimum(base0 + j * CHUNK, n_rows - CHUNK)

    def dma_start(base0, j, buf, sem):
      pltpu.make_async_copy(x_hbm.at[pl.ds(win_start(base0, j), CHUNK)],
                            buf, sem).start()

    def seg_body(s, _):
      off = jnp.maximum(s - 1, 0)
      bv = bnd[pl.ds(off, LANES)]
      is0 = s == 0
      r0 = jnp.maximum(jnp.where(is0, 0, bv[0]), lo)
      r1 = jnp.minimum(jnp.where(is0, bv[0], bv[1]), hi)
      base0 = (r0 // 8) * 8
      nch = jnp.where(r1 > r0,
                      lax.div(r1 - base0 + (CHUNK - 1), CHUNK), 0)

      def process(j, buf, sem, pfbuf, pfsem, accs):
        @pl.when(j + 1 < nch)
        def _():
          dma_start(base0, j + 1, pfbuf, pfsem)

        @pl.when(j < nch)
        def _():
          pltpu.make_async_copy(x_hbm.at[pl.ds(0, CHUNK)], buf, sem).wait()

        w0 = win_start(base0, j)
        a = jnp.maximum(r0, w0) - w0
        b = jnp.where(j < nch, jnp.minimum(r1, w0 + CHUNK) - w0, a)

        def row_body(r, accs):
          return tuple(accs[k] + buf[r, pl.ds(LANES * k, LANES)]
                       for k in range(nlane_blocks))

        return lax.fori_loop(a, b, row_body, accs)

      @pl.when(nch > 0)
      def _():
        dma_start(base0, 0, buf0, sem0)

      def pair_body(p, accs):
        j0 = 2 * p
        accs = process(j0, buf0, sem0, buf1, sem1, accs)
        accs = process(j0 + 1, buf1, sem1, buf0, sem0, accs)
        return accs

      zero = jnp.zeros((LANES,), jnp.float32)
      accs = lax.fori_loop(0, (nch + 1) // 2, pair_body,
                           tuple(zero for _ in range(nlane_blocks)))
      for k in range(nlane_blocks):
        acc2d[s, pl.ds(LANES * k, LANES)] = accs[k]
      return 0

    lax.fori_loop(0, nseg, seg_body, 0)
    pltpu.sync_copy(acc2d, out_hbm.at[sid])

  return seg_sum


def _make_tc_partial(nseg: int):
  def body(x_ref, lab_ref, out_ref):
    @pl.when(pl.program_id(0) == 0)
    def _():
      out_ref[...] = jnp.zeros_like(out_ref)

    lab = lab_ref[...].reshape(1, BLK)
    rows = lax.broadcasted_iota(jnp.int32, (nseg, BLK), 0)
    oh = (rows == lab).astype(jnp.float32)
    out_ref[...] += jnp.dot(oh, x_ref[...],
                            preferred_element_type=jnp.float32)

  return body


def _mlp_body(feats_ref, w1_ref, b1_ref, w2_ref, b2_ref, out_ref):
  h = jnp.dot(feats_ref[...], w1_ref[...],
              preferred_element_type=jnp.float32) + b1_ref[...]
  h = jnp.maximum(h, 0.0)
  logits = jnp.dot(h, w2_ref[...],
                   preferred_element_type=jnp.float32) + b2_ref[...]
  m = jnp.max(logits, axis=-1, keepdims=True)
  e = jnp.exp(logits - m)
  out_ref[...] = e / jnp.sum(e, axis=-1, keepdims=True)


def kernel(x, subject_labels, W1, b1, W2, b2):
  n, d = x.shape
  nseg = b2.shape[0]
  labels = subject_labels.astype(jnp.int32)

  nblk = n // BLK
  nunit = n // SPLIT_UNIT
  split = int(round(SC_FRAC * nunit)) * SPLIT_UNIT
  offb = split // BLK

  partials, bnd_out, _stage = _make_seg_sum_sc(n, d, nseg, split)(x, labels)
  feats_sc = jnp.sum(partials, axis=0)

  feats_tc = pl.pallas_call(
      _make_tc_partial(nseg),
      grid=(nblk - offb,),
      in_specs=[
          pl.BlockSpec((BLK, d), lambda i: (offb + i, 0)),
          pl.BlockSpec((1, 1, BLK), lambda i: (offb + i, 0, 0)),
      ],
      out_specs=pl.BlockSpec((nseg, d), lambda i: (0, 0)),
      out_shape=jax.ShapeDtypeStruct((nseg, d), jnp.float32),
  )(x, labels.reshape(nblk, 1, BLK))

  bounds = jnp.concatenate([jnp.zeros((1,), jnp.int32), bnd_out[:nseg]])
  counts = bounds[1:] - bounds[:-1]
  feats = (feats_sc + feats_tc) / counts[:, None].astype(jnp.float32)

  probs = pl.pallas_call(
      _mlp_body,
      out_shape=jax.ShapeDtypeStruct((nseg, nseg), jnp.float32),
  )(feats, W1, b1.reshape(1, -1), W2, b2.reshape(1, -1))

  vals = jnp.arange(nseg, dtype=jnp.int32)
  present = counts > 0
  k = jnp.sum(present.astype(jnp.int32))
  ranks = jnp.cumsum(present.astype(jnp.int32)) - 1
  tgt = jnp.where(present, ranks, nseg - 1)
  compact = jnp.zeros((nseg,), jnp.int32).at[tgt].set(vals)
  uids = jnp.where(vals < k, compact, compact[0]).astype(subject_labels.dtype)

  return (probs, uids)


# speedup vs baseline: 1.1665x; 1.0540x over previous
import functools

import jax
import jax.numpy as jnp
from jax import lax
from jax.experimental import pallas as pl
from jax.experimental.pallas import tpu as pltpu
from jax.experimental.pallas import tpu_sc as plsc

LANES = 16
CHUNK = 256
NSUB = 16
NW = 32
BLK = 4000
SPLIT_UNIT = 4000
SC_FRAC = 0.575


def _make_seg_sum_sc(n_rows: int, d: int, nseg: int, split: int):
  nlane_blocks = d // LANES
  pt = n_rows // NSUB
  bits = []
  b = 1
  while b <= pt:
    b *= 2
  while b >= 1:
    bits.append(b)
    b //= 2
  mesh = plsc.VectorSubcoreMesh(core_axis_name="c", subcore_axis_name="s")

  @functools.partial(
      pl.kernel,
      out_type=(
          jax.ShapeDtypeStruct((NW, nseg, d), jnp.float32),
          jax.ShapeDtypeStruct((48,), jnp.int32),
          jax.ShapeDtypeStruct((2 * NSUB, 2 * LANES), jnp.int32),
      ),
      mesh=mesh,
      compiler_params=pltpu.CompilerParams(needs_layout_passes=False),
      scratch_types=[
          pltpu.VMEM((pt,), jnp.int32),
          pltpu.VMEM((CHUNK, d), jnp.float32),
          pltpu.VMEM((CHUNK, d), jnp.float32),
          pltpu.VMEM((48,), jnp.int32),
          pltpu.VMEM((2 * LANES,), jnp.int32),
          pltpu.VMEM((NSUB, 2 * LANES), jnp.int32),
          pltpu.VMEM((nseg, d), jnp.float32),
          pltpu.SemaphoreType.DMA,
          pltpu.SemaphoreType.DMA,
      ],
  )
  def seg_sum(x_hbm, labels_hbm, out_hbm, bounds_hbm, stage_hbm,
              lab, buf0, buf1, bnd, cbuf, call, acc2d, sem0, sem1):
    num_cores = jax.lax.axis_size("c")
    cid = lax.axis_index("c")
    tid = lax.axis_index("s")
    sid = tid * num_cores + cid

    pltpu.sync_copy(labels_hbm.at[pl.ds(tid * pt, pt)], lab)
    iota = lax.iota(jnp.int32, LANES)
    q0 = iota + 1
    q1 = iota + 17
    pos0 = jnp.zeros((LANES,), jnp.int32)
    pos1 = jnp.zeros((LANES,), jnp.int32)
    for bit in bits:
      cand0 = pos0 + bit
      cand1 = pos1 + bit
      v0 = plsc.load_gather(lab, [jnp.minimum(cand0, pt) - 1])
      v1 = plsc.load_gather(lab, [jnp.minimum(cand1, pt) - 1])
      pos0 = jnp.where((cand0 <= pt) & (v0 < q0), cand0, pos0)
      pos1 = jnp.where((cand1 <= pt) & (v1 < q1), cand1, pos1)
    cbuf[pl.ds(0, LANES)] = pos0
    cbuf[pl.ds(LANES, LANES)] = pos1
    pltpu.sync_copy(cbuf, stage_hbm.at[cid * NSUB + tid])
    plsc.subcore_barrier()
    pltpu.sync_copy(stage_hbm.at[pl.ds(cid * NSUB, NSUB)], call)
    g0 = jnp.zeros((LANES,), jnp.int32)
    g1 = jnp.zeros((LANES,), jnp.int32)
    for i in range(NSUB):
      g0 = g0 + call[i, pl.ds(0, LANES)]
      g1 = g1 + call[i, pl.ds(LANES, LANES)]
    bnd[pl.ds(0, LANES)] = g0
    bnd[pl.ds(LANES, LANES)] = g1

    @pl.when(sid == 0)
    def _():
      pltpu.sync_copy(bnd, bounds_hbm)

    share = split // NW
    lo = sid * share
    hi = lo + share
    s_first = (jnp.sum((g0 <= lo).astype(jnp.int32))
               + jnp.sum((g1 <= lo).astype(jnp.int32)))
    s_last = (jnp.sum((g0 <= hi - 1).astype(jnp.int32))
              + jnp.sum((g1 <= hi - 1).astype(jnp.int32)))

    def win_start(base0, j):
      return jnp.minimum(base0 + j * CHUNK, n_rows - CHUNK)

    def dma_start(base0, j, buf, sem):
      pltpu.make_async_copy(x_hbm.at[pl.ds(win_start(base0, j), CHUNK)],
                            buf, sem).start()

    zerov = jnp.zeros((LANES,), jnp.float32)

    def zero_body(s, _):
      for k in range(nlane_blocks):
        acc2d[s, pl.ds(LANES * k, LANES)] = zerov
      return 0

    lax.fori_loop(0, nseg, zero_body, 0)

    def seg_body(s, _):
      off = jnp.maximum(s - 1, 0)
      bv = bnd[pl.ds(off, LANES)]
      is0 = s == 0
      r0 = jnp.maximum(jnp.where(is0, 0, bv[0]), lo)
      r1 = jnp.minimum(jnp.where(is0, bv[0], bv[1]), hi)
      base0 = (r0 // 8) * 8
      nch = jnp.where(r1 > r0,
                      lax.div(r1 - base0 + (CHUNK - 1), CHUNK), 0)

      def process(j, buf, sem, pfbuf, pfsem, accs):
        @pl.when(j + 1 < nch)
        def _():
          dma_start(base0, j + 1, pfbuf, pfsem)

        @pl.when(j < nch)
        def _():
          pltpu.make_async_copy(x_hbm.at[pl.ds(0, CHUNK)], buf, sem).wait()

        w0 = win_start(base0, j)
        a = jnp.maximum(r0, w0) - w0
        b = jnp.where(j < nch, jnp.minimum(r1, w0 + CHUNK) - w0, a)

        def row_body(r, accs):
          return tuple(accs[k] + buf[r, pl.ds(LANES * k, LANES)]
                       for k in range(nlane_blocks))

        return lax.fori_loop(a, b, row_body, accs)

      @pl.when(nch > 0)
      def _():
        dma_start(base0, 0, buf0, sem0)

      def pair_body(p, accs):
        j0 = 2 * p
        accs = process(j0, buf0, sem0, buf1, sem1, accs)
        accs = process(j0 + 1, buf1, sem1, buf0, sem0, accs)
        return accs

      zero = jnp.zeros((LANES,), jnp.float32)
      accs = lax.fori_loop(0, (nch + 1) // 2, pair_body,
                           tuple(zero for _ in range(nlane_blocks)))
      for k in range(nlane_blocks):
        acc2d[s, pl.ds(LANES * k, LANES)] = accs[k]
      return 0

    lax.fori_loop(s_first, s_last + 1, seg_body, 0)
    pltpu.sync_copy(acc2d, out_hbm.at[sid])

  return seg_sum


def _make_tc_partial(nseg: int):
  def body(x_ref, lab_ref, out_ref):
    @pl.when(pl.program_id(0) == 0)
    def _():
      out_ref[...] = jnp.zeros_like(out_ref)

    lab = lab_ref[...].reshape(1, BLK)
    rows = lax.broadcasted_iota(jnp.int32, (nseg, BLK), 0)
    oh = (rows == lab).astype(jnp.float32)
    out_ref[...] += jnp.dot(oh, x_ref[...],
                            preferred_element_type=jnp.float32)

  return body


def _mlp_body(feats_ref, w1_ref, b1_ref, w2_ref, b2_ref, out_ref):
  h = jnp.dot(feats_ref[...], w1_ref[...],
              preferred_element_type=jnp.float32) + b1_ref[...]
  h = jnp.maximum(h, 0.0)
  logits = jnp.dot(h, w2_ref[...],
                   preferred_element_type=jnp.float32) + b2_ref[...]
  m = jnp.max(logits, axis=-1, keepdims=True)
  e = jnp.exp(logits - m)
  out_ref[...] = e / jnp.sum(e, axis=-1, keepdims=True)


def kernel(x, subject_labels, W1, b1, W2, b2):
  n, d = x.shape
  nseg = b2.shape[0]
  labels = subject_labels.astype(jnp.int32)

  nblk = n // BLK
  nunit = n // SPLIT_UNIT
  split = int(round(SC_FRAC * nunit)) * SPLIT_UNIT
  offb = split // BLK

  partials, bnd_out, _stage = _make_seg_sum_sc(n, d, nseg, split)(x, labels)
  feats_sc = jnp.sum(partials, axis=0)

  feats_tc = pl.pallas_call(
      _make_tc_partial(nseg),
      grid=(nblk - offb,),
      in_specs=[
          pl.BlockSpec((BLK, d), lambda i: (offb + i, 0)),
          pl.BlockSpec((1, 8, BLK // 8), lambda i: (offb + i, 0, 0)),
      ],
      out_specs=pl.BlockSpec((nseg, d), lambda i: (0, 0)),
      out_shape=jax.ShapeDtypeStruct((nseg, d), jnp.float32),
  )(x, labels.reshape(nblk, 8, BLK // 8))

  bounds = jnp.concatenate([jnp.zeros((1,), jnp.int32), bnd_out[:nseg]])
  counts = bounds[1:] - bounds[:-1]
  feats = (feats_sc + feats_tc) / counts[:, None].astype(jnp.float32)

  probs = pl.pallas_call(
      _mlp_body,
      out_shape=jax.ShapeDtypeStruct((nseg, nseg), jnp.float32),
  )(feats, W1, b1.reshape(1, -1), W2, b2.reshape(1, -1))

  vals = jnp.arange(nseg, dtype=jnp.int32)
  present = counts > 0
  k = jnp.sum(present.astype(jnp.int32))
  ranks = jnp.cumsum(present.astype(jnp.int32)) - 1
  tgt = jnp.where(present, ranks, nseg - 1)
  compact = jnp.zeros((nseg,), jnp.int32).at[tgt].set(vals)
  uids = jnp.where(vals < k, compact, compact[0]).astype(subject_labels.dtype)

  return (probs, uids)


# speedup vs baseline: 1.1953x; 1.0247x over previous
import functools

import jax
import jax.numpy as jnp
from jax import lax
from jax.experimental import pallas as pl
from jax.experimental.pallas import tpu as pltpu
from jax.experimental.pallas import tpu_sc as plsc

LANES = 16
CHUNK = 384
NSUB = 16
NW = 32
BLK = 4000
SPLIT_UNIT = 4000
SC_FRAC = 0.55


def _make_seg_sum_sc(n_rows: int, d: int, nseg: int, split: int):
  nlane_blocks = d // LANES
  pt = n_rows // NSUB
  bits = []
  b = 1
  while b <= pt:
    b *= 2
  while b >= 1:
    bits.append(b)
    b //= 2
  mesh = plsc.VectorSubcoreMesh(core_axis_name="c", subcore_axis_name="s")

  @functools.partial(
      pl.kernel,
      out_type=(
          jax.ShapeDtypeStruct((NW, nseg, d), jnp.float32),
          jax.ShapeDtypeStruct((48,), jnp.int32),
          jax.ShapeDtypeStruct((2 * NSUB, 2 * LANES), jnp.int32),
      ),
      mesh=mesh,
      compiler_params=pltpu.CompilerParams(needs_layout_passes=False),
      scratch_types=[
          pltpu.VMEM((pt,), jnp.int32),
          pltpu.VMEM((CHUNK, d), jnp.float32),
          pltpu.VMEM((CHUNK, d), jnp.float32),
          pltpu.VMEM((48,), jnp.int32),
          pltpu.VMEM((2 * LANES,), jnp.int32),
          pltpu.VMEM((NSUB, 2 * LANES), jnp.int32),
          pltpu.VMEM((nseg, d), jnp.float32),
          pltpu.SemaphoreType.DMA,
          pltpu.SemaphoreType.DMA,
      ],
  )
  def seg_sum(x_hbm, labels_hbm, out_hbm, bounds_hbm, stage_hbm,
              lab, buf0, buf1, bnd, cbuf, call, acc2d, sem0, sem1):
    num_cores = jax.lax.axis_size("c")
    cid = lax.axis_index("c")
    tid = lax.axis_index("s")
    sid = tid * num_cores + cid

    pltpu.sync_copy(labels_hbm.at[pl.ds(tid * pt, pt)], lab)
    iota = lax.iota(jnp.int32, LANES)
    q0 = iota + 1
    q1 = iota + 17
    pos0 = jnp.zeros((LANES,), jnp.int32)
    pos1 = jnp.zeros((LANES,), jnp.int32)
    for bit in bits:
      cand0 = pos0 + bit
      cand1 = pos1 + bit
      v0 = plsc.load_gather(lab, [jnp.minimum(cand0, pt) - 1])
      v1 = plsc.load_gather(lab, [jnp.minimum(cand1, pt) - 1])
      pos0 = jnp.where((cand0 <= pt) & (v0 < q0), cand0, pos0)
      pos1 = jnp.where((cand1 <= pt) & (v1 < q1), cand1, pos1)
    cbuf[pl.ds(0, LANES)] = pos0
    cbuf[pl.ds(LANES, LANES)] = pos1
    pltpu.sync_copy(cbuf, stage_hbm.at[cid * NSUB + tid])
    plsc.subcore_barrier()
    pltpu.sync_copy(stage_hbm.at[pl.ds(cid * NSUB, NSUB)], call)
    g0 = jnp.zeros((LANES,), jnp.int32)
    g1 = jnp.zeros((LANES,), jnp.int32)
    for i in range(NSUB):
      g0 = g0 + call[i, pl.ds(0, LANES)]
      g1 = g1 + call[i, pl.ds(LANES, LANES)]
    bnd[pl.ds(0, LANES)] = g0
    bnd[pl.ds(LANES, LANES)] = g1

    @pl.when(sid == 0)
    def _():
      pltpu.sync_copy(bnd, bounds_hbm)

    share = split // NW
    lo = sid * share
    hi = lo + share
    s_first = (jnp.sum((g0 <= lo).astype(jnp.int32))
               + jnp.sum((g1 <= lo).astype(jnp.int32)))
    s_last = (jnp.sum((g0 <= hi - 1).astype(jnp.int32))
              + jnp.sum((g1 <= hi - 1).astype(jnp.int32)))

    def win_start(base0, j):
      return jnp.minimum(base0 + j * CHUNK, n_rows - CHUNK)

    def dma_start(base0, j, buf, sem):
      pltpu.make_async_copy(x_hbm.at[pl.ds(win_start(base0, j), CHUNK)],
                            buf, sem).start()

    zerov = jnp.zeros((LANES,), jnp.float32)

    def zero_body(s, _):
      for k in range(nlane_blocks):
        acc2d[s, pl.ds(LANES * k, LANES)] = zerov
      return 0

    lax.fori_loop(0, nseg, zero_body, 0)

    def seg_body(s, _):
      off = jnp.maximum(s - 1, 0)
      bv = bnd[pl.ds(off, LANES)]
      is0 = s == 0
      r0 = jnp.maximum(jnp.where(is0, 0, bv[0]), lo)
      r1 = jnp.minimum(jnp.where(is0, bv[0], bv[1]), hi)
      base0 = (r0 // 8) * 8
      nch = jnp.where(r1 > r0,
                      lax.div(r1 - base0 + (CHUNK - 1), CHUNK), 0)

      def process(j, buf, sem, pfbuf, pfsem, accs):
        @pl.when(j + 1 < nch)
        def _():
          dma_start(base0, j + 1, pfbuf, pfsem)

        @pl.when(j < nch)
        def _():
          pltpu.make_async_copy(x_hbm.at[pl.ds(0, CHUNK)], buf, sem).wait()

        w0 = win_start(base0, j)
        a = jnp.maximum(r0, w0) - w0
        b = jnp.where(j < nch, jnp.minimum(r1, w0 + CHUNK) - w0, a)

        def row_body(r, accs):
          return tuple(accs[k] + buf[r, pl.ds(LANES * k, LANES)]
                       for k in range(nlane_blocks))

        return lax.fori_loop(a, b, row_body, accs)

      @pl.when(nch > 0)
      def _():
        dma_start(base0, 0, buf0, sem0)

      def pair_body(p, accs):
        j0 = 2 * p
        accs = process(j0, buf0, sem0, buf1, sem1, accs)
        accs = process(j0 + 1, buf1, sem1, buf0, sem0, accs)
        return accs

      zero = jnp.zeros((LANES,), jnp.float32)
      accs = lax.fori_loop(0, (nch + 1) // 2, pair_body,
                           tuple(zero for _ in range(nlane_blocks)))
      for k in range(nlane_blocks):
        acc2d[s, pl.ds(LANES * k, LANES)] = accs[k]
      return 0

    lax.fori_loop(s_first, s_last + 1, seg_body, 0)
    pltpu.sync_copy(acc2d, out_hbm.at[sid])

  return seg_sum


def _make_tc_partial(nseg: int):
  def body(x_ref, lab_ref, out_ref):
    @pl.when(pl.program_id(0) == 0)
    def _():
      out_ref[...] = jnp.zeros_like(out_ref)

    lab = lab_ref[...].reshape(1, BLK)
    rows = lax.broadcasted_iota(jnp.int32, (nseg, BLK), 0)
    oh = (rows == lab).astype(jnp.float32)
    out_ref[...] += jnp.dot(oh, x_ref[...],
                            preferred_element_type=jnp.float32)

  return body


def _make_fused_tail(nseg: int):
  def body(partials_ref, ftc_ref, bndc_ref, bndr_ref,
           w1_ref, b1_ref, w2_ref, b2_ref, probs_ref, uids_ref):
    s = ftc_ref[...]
    for w in range(NW):
      s = s + partials_ref[w]

    fdt = jnp.float32
    iota_r = lax.broadcasted_iota(jnp.int32, (nseg, nseg), 0)
    iota_c = lax.broadcasted_iota(jnp.int32, (nseg, nseg), 1)

    b_col = bndc_ref[...][:nseg, :].astype(fdt)
    shift_col = (iota_c == iota_r - 1).astype(fdt)
    prev_col = jnp.dot(shift_col, b_col, preferred_element_type=fdt)
    cnt_col = b_col - prev_col

    feats = s / cnt_col
    h = jnp.dot(feats, w1_ref[...], preferred_element_type=fdt) + b1_ref[...]
    h = jnp.maximum(h, 0.0)
    logits = jnp.dot(h, w2_ref[...], preferred_element_type=fdt) + b2_ref[...]
    m = jnp.max(logits, axis=-1, keepdims=True)
    e = jnp.exp(logits - m)
    probs_ref[...] = e / jnp.sum(e, axis=-1, keepdims=True)

    b_row = bndr_ref[...][:, :nseg].astype(fdt)
    shift_row = (iota_r == iota_c - 1).astype(fdt)
    prev_row = jnp.dot(b_row, shift_row, preferred_element_type=fdt)
    cnt_row = b_row - prev_row
    present = (cnt_row > 0).astype(fdt)
    tril_t = (iota_r <= iota_c).astype(fdt)
    cums = jnp.dot(present, tril_t, preferred_element_type=fdt)
    ranks = cums - 1.0
    k = jnp.sum(present)
    vals_row = lax.broadcasted_iota(jnp.int32, (1, nseg), 1).astype(fdt)
    minp = jnp.min(jnp.where(present > 0, vals_row, fdt(nseg)))
    oh_t = ((iota_r.astype(fdt) == ranks) * present)
    vals_col = lax.broadcasted_iota(jnp.int32, (nseg, 1), 0).astype(fdt)
    uids_col = jnp.dot(oh_t, vals_col, preferred_element_type=fdt)
    pos_col = vals_col
    uids_ref[...] = jnp.where(pos_col < k, uids_col, minp).astype(jnp.int32)

  return body


def kernel(x, subject_labels, W1, b1, W2, b2):
  n, d = x.shape
  nseg = b2.shape[0]
  labels = subject_labels.astype(jnp.int32)

  nblk = n // BLK
  nunit = n // SPLIT_UNIT
  split = int(round(SC_FRAC * nunit)) * SPLIT_UNIT
  offb = split // BLK

  partials, bnd_out, _stage = _make_seg_sum_sc(n, d, nseg, split)(x, labels)

  feats_tc = pl.pallas_call(
      _make_tc_partial(nseg),
      grid=(nblk - offb,),
      in_specs=[
          pl.BlockSpec((BLK, d), lambda i: (offb + i, 0)),
          pl.BlockSpec((1, 8, BLK // 8), lambda i: (offb + i, 0, 0)),
      ],
      out_specs=pl.BlockSpec((nseg, d), lambda i: (0, 0)),
      out_shape=jax.ShapeDtypeStruct((nseg, d), jnp.float32),
  )(x, labels.reshape(nblk, 8, BLK // 8))

  probs, uids2d = pl.pallas_call(
      _make_fused_tail(nseg),
      out_shape=(
          jax.ShapeDtypeStruct((nseg, nseg), jnp.float32),
          jax.ShapeDtypeStruct((nseg, 1), jnp.int32),
      ),
  )(partials, feats_tc, bnd_out.reshape(-1, 1), bnd_out.reshape(1, -1),
    W1, b1.reshape(1, -1), W2, b2.reshape(1, -1))

  uids = uids2d.reshape(nseg).astype(subject_labels.dtype)
  return (probs, uids)


# speedup vs baseline: 1.2165x; 1.0177x over previous
import functools

import jax
import jax.numpy as jnp
from jax import lax
from jax.experimental import pallas as pl
from jax.experimental.pallas import tpu as pltpu
from jax.experimental.pallas import tpu_sc as plsc

LANES = 16
CHUNK = 384
NSUB = 16
NW = 32
BLK = 4000
SPLIT_UNIT = 4000
SC_FRAC = 0.5375


def _make_seg_sum_sc(n_rows: int, d: int, nseg: int, split: int):
  nlane_blocks = d // LANES
  pt = n_rows // NSUB
  bits = []
  b = 1
  while b <= pt:
    b *= 2
  while b >= 1:
    bits.append(b)
    b //= 2
  mesh = plsc.VectorSubcoreMesh(core_axis_name="c", subcore_axis_name="s")

  @functools.partial(
      pl.kernel,
      out_type=(
          jax.ShapeDtypeStruct((NW, nseg, d), jnp.float32),
          jax.ShapeDtypeStruct((48,), jnp.int32),
          jax.ShapeDtypeStruct((2 * NSUB, 2 * LANES), jnp.int32),
      ),
      mesh=mesh,
      compiler_params=pltpu.CompilerParams(needs_layout_passes=False),
      scratch_types=[
          pltpu.VMEM((pt,), jnp.int32),
          pltpu.VMEM((CHUNK, d), jnp.float32),
          pltpu.VMEM((CHUNK, d), jnp.float32),
          pltpu.VMEM((48,), jnp.int32),
          pltpu.VMEM((2 * LANES,), jnp.int32),
          pltpu.VMEM((NSUB, 2 * LANES), jnp.int32),
          pltpu.VMEM((nseg, d), jnp.float32),
          pltpu.SemaphoreType.DMA,
          pltpu.SemaphoreType.DMA,
      ],
  )
  def seg_sum(x_hbm, labels_hbm, out_hbm, bounds_hbm, stage_hbm,
              lab, buf0, buf1, bnd, cbuf, call, acc2d, sem0, sem1):
    num_cores = jax.lax.axis_size("c")
    cid = lax.axis_index("c")
    tid = lax.axis_index("s")
    sid = tid * num_cores + cid

    pltpu.sync_copy(labels_hbm.at[pl.ds(tid * pt, pt)], lab)
    iota = lax.iota(jnp.int32, LANES)
    q0 = iota + 1
    q1 = iota + 17
    pos0 = jnp.zeros((LANES,), jnp.int32)
    pos1 = jnp.zeros((LANES,), jnp.int32)
    for bit in bits:
      cand0 = pos0 + bit
      cand1 = pos1 + bit
      v0 = plsc.load_gather(lab, [jnp.minimum(cand0, pt) - 1])
      v1 = plsc.load_gather(lab, [jnp.minimum(cand1, pt) - 1])
      pos0 = jnp.where((cand0 <= pt) & (v0 < q0), cand0, pos0)
      pos1 = jnp.where((cand1 <= pt) & (v1 < q1), cand1, pos1)
    cbuf[pl.ds(0, LANES)] = pos0
    cbuf[pl.ds(LANES, LANES)] = pos1
    pltpu.sync_copy(cbuf, stage_hbm.at[cid * NSUB + tid])
    plsc.subcore_barrier()
    pltpu.sync_copy(stage_hbm.at[pl.ds(cid * NSUB, NSUB)], call)
    g0 = jnp.zeros((LANES,), jnp.int32)
    g1 = jnp.zeros((LANES,), jnp.int32)
    for i in range(NSUB):
      g0 = g0 + call[i, pl.ds(0, LANES)]
      g1 = g1 + call[i, pl.ds(LANES, LANES)]
    bnd[pl.ds(0, LANES)] = g0
    bnd[pl.ds(LANES, LANES)] = g1

    @pl.when(sid == 0)
    def _():
      pltpu.sync_copy(bnd, bounds_hbm)

    share = split // NW
    lo = sid * share
    hi = lo + share
    s_first = (jnp.sum((g0 <= lo).astype(jnp.int32))
               + jnp.sum((g1 <= lo).astype(jnp.int32)))
    s_last = (jnp.sum((g0 <= hi - 1).astype(jnp.int32))
              + jnp.sum((g1 <= hi - 1).astype(jnp.int32)))

    def win_start(base0, j):
      return jnp.minimum(base0 + j * CHUNK, n_rows - CHUNK)

    def dma_start(base0, j, buf, sem):
      pltpu.make_async_copy(x_hbm.at[pl.ds(win_start(base0, j), CHUNK)],
                            buf, sem).start()

    zerov = jnp.zeros((LANES,), jnp.float32)

    def zero_body(s, _):
      for k in range(nlane_blocks):
        acc2d[s, pl.ds(LANES * k, LANES)] = zerov
      return 0

    lax.fori_loop(0, nseg, zero_body, 0)

    def seg_body(s, _):
      off = jnp.maximum(s - 1, 0)
      bv = bnd[pl.ds(off, LANES)]
      is0 = s == 0
      r0 = jnp.maximum(jnp.where(is0, 0, bv[0]), lo)
      r1 = jnp.minimum(jnp.where(is0, bv[0], bv[1]), hi)
      base0 = (r0 // 8) * 8
      nch = jnp.where(r1 > r0,
                      lax.div(r1 - base0 + (CHUNK - 1), CHUNK), 0)

      def process(j, buf, sem, pfbuf, pfsem, accs):
        @pl.when(j + 1 < nch)
        def _():
          dma_start(base0, j + 1, pfbuf, pfsem)

        @pl.when(j < nch)
        def _():
          pltpu.make_async_copy(x_hbm.at[pl.ds(0, CHUNK)], buf, sem).wait()

        w0 = win_start(base0, j)
        a = jnp.maximum(r0, w0) - w0
        b = jnp.where(j < nch, jnp.minimum(r1, w0 + CHUNK) - w0, a)

        def row_body(r, accs):
          return tuple(accs[k] + buf[r, pl.ds(LANES * k, LANES)]
                       for k in range(nlane_blocks))

        return lax.fori_loop(a, b, row_body, accs)

      @pl.when(nch > 0)
      def _():
        dma_start(base0, 0, buf0, sem0)

      def pair_body(p, accs):
        j0 = 2 * p
        accs = process(j0, buf0, sem0, buf1, sem1, accs)
        accs = process(j0 + 1, buf1, sem1, buf0, sem0, accs)
        return accs

      zero = jnp.zeros((LANES,), jnp.float32)
      accs = lax.fori_loop(0, (nch + 1) // 2, pair_body,
                           tuple(zero for _ in range(nlane_blocks)))
      for k in range(nlane_blocks):
        acc2d[s, pl.ds(LANES * k, LANES)] = accs[k]
      return 0

    lax.fori_loop(s_first, s_last + 1, seg_body, 0)
    pltpu.sync_copy(acc2d, out_hbm.at[sid])

  return seg_sum


def _make_tc_partial(nseg: int):
  def body(x_ref, lab_ref, out_ref):
    @pl.when(pl.program_id(0) == 0)
    def _():
      out_ref[...] = jnp.zeros_like(out_ref)

    lab = lab_ref[...].reshape(1, BLK)
    rows = lax.broadcasted_iota(jnp.int32, (nseg, BLK), 0)
    oh = (rows == lab).astype(jnp.float32)
    out_ref[...] += jnp.dot(oh, x_ref[...],
                            preferred_element_type=jnp.float32)

  return body


def _make_fused_tail(nseg: int):
  def body(partials_ref, ftc_ref, bndc_ref, bndr_ref,
           w1_ref, b1_ref, w2_ref, b2_ref, probs_ref, uids_ref):
    s = ftc_ref[...]
    for w in range(NW):
      s = s + partials_ref[w]

    fdt = jnp.float32
    iota_r = lax.broadcasted_iota(jnp.int32, (nseg, nseg), 0)
    iota_c = lax.broadcasted_iota(jnp.int32, (nseg, nseg), 1)

    b_col = bndc_ref[...][:nseg, :].astype(fdt)
    shift_col = (iota_c == iota_r - 1).astype(fdt)
    prev_col = jnp.dot(shift_col, b_col, preferred_element_type=fdt)
    cnt_col = b_col - prev_col

    feats = s / cnt_col
    h = jnp.dot(feats, w1_ref[...], preferred_element_type=fdt) + b1_ref[...]
    h = jnp.maximum(h, 0.0)
    logits = jnp.dot(h, w2_ref[...], preferred_element_type=fdt) + b2_ref[...]
    m = jnp.max(logits, axis=-1, keepdims=True)
    e = jnp.exp(logits - m)
    probs_ref[...] = e / jnp.sum(e, axis=-1, keepdims=True)

    b_row = bndr_ref[...][:, :nseg].astype(fdt)
    shift_row = (iota_r == iota_c - 1).astype(fdt)
    prev_row = jnp.dot(b_row, shift_row, preferred_element_type=fdt)
    cnt_row = b_row - prev_row
    present = (cnt_row > 0).astype(fdt)
    tril_t = (iota_r <= iota_c).astype(fdt)
    cums = jnp.dot(present, tril_t, preferred_element_type=fdt)
    ranks = cums - 1.0
    k = jnp.sum(present)
    vals_row = lax.broadcasted_iota(jnp.int32, (1, nseg), 1).astype(fdt)
    minp = jnp.min(jnp.where(present > 0, vals_row, fdt(nseg)))
    oh_t = ((iota_r.astype(fdt) == ranks) * present)
    vals_col = lax.broadcasted_iota(jnp.int32, (nseg, 1), 0).astype(fdt)
    uids_col = jnp.dot(oh_t, vals_col, preferred_element_type=fdt)
    pos_col = vals_col
    uids_ref[...] = jnp.where(pos_col < k, uids_col, minp).astype(jnp.int32)

  return body


def kernel(x, subject_labels, W1, b1, W2, b2):
  n, d = x.shape
  nseg = b2.shape[0]
  labels = subject_labels.astype(jnp.int32)

  nblk = n // BLK
  nunit = n // SPLIT_UNIT
  split = int(round(SC_FRAC * nunit)) * SPLIT_UNIT
  offb = split // BLK

  partials, bnd_out, _stage = _make_seg_sum_sc(n, d, nseg, split)(x, labels)

  feats_tc = pl.pallas_call(
      _make_tc_partial(nseg),
      grid=(nblk - offb,),
      in_specs=[
          pl.BlockSpec((BLK, d), lambda i: (offb + i, 0)),
          pl.BlockSpec((1, 8, BLK // 8), lambda i: (offb + i, 0, 0)),
      ],
      out_specs=pl.BlockSpec((nseg, d), lambda i: (0, 0)),
      out_shape=jax.ShapeDtypeStruct((nseg, d), jnp.float32),
  )(x, labels.reshape(nblk, 8, BLK // 8))

  probs, uids2d = pl.pallas_call(
      _make_fused_tail(nseg),
      out_shape=(
          jax.ShapeDtypeStruct((nseg, nseg), jnp.float32),
          jax.ShapeDtypeStruct((nseg, 1), jnp.int32),
      ),
  )(partials, feats_tc, bnd_out.reshape(-1, 1), bnd_out.reshape(1, -1),
    W1, b1.reshape(1, -1), W2, b2.reshape(1, -1))

  uids = uids2d.reshape(nseg).astype(subject_labels.dtype)
  return (probs, uids)
